# baseline (speedup 1.0000x reference)
import dataclasses
import functools

import jax
import jax.numpy as jnp
from jax import lax
from jax.experimental import pallas as pl
from jax.experimental.pallas import tpu as pltpu
from jax.experimental.pallas import tpu_sc as plsc

N = 16384
D_IN = 2048
D_H = 1024
TILE_M = 256
TILE_E = 512
R = 128
NW = 32
ROWS_PW = R // NW
RPW = TILE_E
CH = 16
MAXC_PW = RPW // CH


def _mesh():
    return plsc.VectorSubcoreMesh(core_axis_name="c", subcore_axis_name="s")


def _sc_params():
    cp = pltpu.CompilerParams()
    if "needs_layout_passes" in pltpu.CompilerParams.__dataclass_fields__:
        cp = dataclasses.replace(cp, needs_layout_passes=False)
    return cp


def _dot(a, b):
    return lax.dot_general(a, b, (((1,), (0,)), ((), ())),
                           preferred_element_type=jnp.float32)


def _route_body(lv_ref, dst_ref, k1_ref, opad_ref):
    lv = lv_ref[...]
    m = (lv == 0).astype(jnp.float32)
    r = lax.broadcasted_iota(jnp.int32, (R, R), 0)
    c = lax.broadcasted_iota(jnp.int32, (R, R), 1)
    upper = (r < c).astype(jnp.float32)
    lower = (c < r).astype(jnp.float32)
    e = _dot(m, upper)
    s = e[:, R - 1:R] + m[:, R - 1:R]
    off = _dot(lower, s)

    wg = lax.broadcasted_iota(jnp.int32, (NW, R), 0)
    rg = lax.broadcasted_iota(jnp.int32, (NW, R), 1)
    grp = (rg // ROWS_PW == wg).astype(jnp.float32)
    cw = (_dot(grp, s) + 0.5).astype(jnp.int32)
    cpad16 = ((cw + 15) // 16).astype(jnp.float32)
    kpad = 16 * jnp.sum(cpad16).astype(jnp.int32)

    rw = lax.broadcasted_iota(jnp.int32, (R, NW), 0)
    ww = lax.broadcasted_iota(jnp.int32, (R, NW), 1)
    expl = (ww < rw // ROWS_PW).astype(jnp.float32)
    opad128 = 16.0 * _dot(expl, cpad16)
    lower4 = (c < (r // ROWS_PW) * ROWS_PW).astype(jnp.float32)
    off4 = _dot(lower4, s)
    dd = (opad128 - off4 + 0.5).astype(jnp.int32)
    pos = (e + off + 0.5).astype(jnp.int32) + dd
    dst_ref[...] = jnp.where(lv == 0, pos, N)
    k1_ref[0] = kpad

    w1 = lax.broadcasted_iota(jnp.int32, (NW, NW), 0)
    w2 = lax.broadcasted_iota(jnp.int32, (NW, NW), 1)
    lw32 = (w2 < w1).astype(jnp.float32)
    opad_ref[...] = (16.0 * _dot(lw32, cpad16) + 0.5).astype(jnp.int32)


def _route(lv2d):
    return pl.pallas_call(
        _route_body,
        out_shape=(
            jax.ShapeDtypeStruct((R, R), jnp.int32),
            jax.ShapeDtypeStruct((1,), jnp.int32),
            jax.ShapeDtypeStruct((NW, 1), jnp.int32),
        ),
        out_specs=(
            pl.BlockSpec((R, R), lambda: (0, 0)),
            pl.BlockSpec(memory_space=pltpu.SMEM),
            pl.BlockSpec((NW, 1), lambda: (0, 0)),
        ),
    )(lv2d)


def _gather_rows(lv1d, x):
  f = functools.partial(
      pl.kernel,
      out_type=jax.ShapeDtypeStruct((N, D_IN), jnp.float32),
      mesh=_mesh(),
      compiler_params=_sc_params(),
      scratch_types=[
          pltpu.VMEM((N,), jnp.int32),
          pltpu.VMEM((RPW,), jnp.int32),
          pltpu.VMEM((CH,), jnp.int32),
          pltpu.VMEM((CH,), jnp.int32),
          pltpu.VMEM((CH, D_IN), jnp.float32),
          pltpu.VMEM((CH, D_IN), jnp.float32),
          pltpu.SemaphoreType.DMA,
          pltpu.SemaphoreType.DMA,
          pltpu.SemaphoreType.DMA,
          pltpu.SemaphoreType.DMA,
      ],
  )

  @f
  def body(lv_hbm, x_hbm, xc_hbm, lvall, lw, iv0, iv1, rows0,
           rows1, sg0, sg1, sw0, sw1):
    wid = lax.axis_index("s") * 2 + lax.axis_index("c")
    pltpu.sync_copy(lv_hbm, lvall)
    lane = lax.iota(jnp.int32, 16)

    def outer(g, p16acc):
        def inner(j, acc):
            v = lvall[pl.ds(g * RPW + j * 16, 16)]
            return acc + (v == 0).astype(jnp.int32)
        acc = lax.fori_loop(0, RPW // 16, inner, jnp.zeros((16,), jnp.int32))
        cg = jnp.sum(acc, axis=0)
        return p16acc + (cg + 15) // 16

    p16 = lax.fori_loop(0, wid, outer, jnp.int32(0))

    def grp(g, cnt):
        d16 = lvall[pl.ds(wid * RPW + g * 16, 16)]
        msk = d16 == 0
        ids = wid * RPW + g * 16 + lane
        plsc.store_compressed(lw.at[pl.ds(cnt, 16)], ids, mask=msk)
        return cnt + jnp.sum(msk.astype(jnp.int32), axis=0)

    cnt = lax.fori_loop(0, 32, grp, jnp.int32(0))
    ivs = (iv0, iv1)
    rows = (rows0, rows1)
    sgs = (sg0, sg1)
    sws = (sw0, sw1)

    @pl.loop(0, MAXC_PW, step=2)
    def _pair(cl):
        for b in range(2):
            @pl.when((cl + b) * CH < cnt)
            def _(b=b):
                c = cl + b
                ivs[b][...] = plsc.load_gather(
                    lw, [jnp.minimum(c * CH + lane, cnt - 1)])
                pltpu.make_async_copy(x_hbm.at[ivs[b]], rows[b],
                                      sgs[b]).start()
        for b in range(2):
            @pl.when((cl + b) * CH < cnt)
            def _(b=b):
                c = cl + b
                pltpu.make_async_copy(x_hbm.at[ivs[b]], rows[b],
                                      sgs[b]).wait()
                pltpu.make_async_copy(
                    rows[b], xc_hbm.at[pl.ds((p16 + c) * CH, CH)],
                    sws[b]).start()
        for b in range(2):
            @pl.when((cl + b) * CH < cnt)
            def _(b=b):
                c = cl + b
                pltpu.make_async_copy(
                    rows[b], xc_hbm.at[pl.ds((p16 + c) * CH, CH)],
                    sws[b]).wait()

  return body(lv1d, x)


def _mlp_body(k_ref, x_ref, w1_ref, b1_ref, w2_ref, b2_ref, w3_ref, b3_ref,
              o_ref):
    nt = (k_ref[0] + TILE_M - 1) // TILE_M

    @pl.when(pl.program_id(0) < nt)
    def _():
        xb = x_ref[...].astype(jnp.bfloat16)
        h1 = lax.dot_general(xb, w1_ref[...], (((1,), (0,)), ((), ())),
                             preferred_element_type=jnp.float32)
        h1 = jnp.maximum(h1 + b1_ref[...][None, :], 0.0).astype(jnp.bfloat16)
        h2 = lax.dot_general(h1, w2_ref[...], (((1,), (0,)), ((), ())),
                             preferred_element_type=jnp.float32)
        h2 = jnp.maximum(h2 + b2_ref[...][None, :], 0.0)
        out = lax.dot_general(h2, w3_ref[...], (((1,), (0,)), ((), ())),
                              preferred_element_type=jnp.float32)
        o_ref[...] = out + b3_ref[...][None, :]


def _mlp(k1, xc, w1b, b1, w2b, b2, W3, b3):
    def _x_map(i, kr):
        nt = (kr[0] + TILE_M - 1) // TILE_M
        return (jnp.minimum(i, jnp.maximum(nt - 1, 0)), 0)

    grid_spec = pltpu.PrefetchScalarGridSpec(
        num_scalar_prefetch=1,
        grid=(N // TILE_M,),
        in_specs=[
            pl.BlockSpec((TILE_M, D_IN), _x_map),
            pl.BlockSpec((D_IN, D_H), lambda i, kr: (0, 0)),
            pl.BlockSpec((D_H,), lambda i, kr: (0,)),
            pl.BlockSpec((D_H, D_H), lambda i, kr: (0, 0)),
            pl.BlockSpec((D_H,), lambda i, kr: (0,)),
            pl.BlockSpec((D_H, 1), lambda i, kr: (0, 0)),
            pl.BlockSpec((1,), lambda i, kr: (0,)),
        ],
        out_specs=pl.BlockSpec((TILE_M, 1), lambda i, kr: (i, 0)),
    )
    return pl.pallas_call(
        _mlp_body,
        grid_spec=grid_spec,
        out_shape=jax.ShapeDtypeStruct((N + TILE_E, 1), jnp.float32),
    )(k1, xc, w1b, b1, w2b, b2, W3, b3)


def _expand_body(tb_ref, dst_ref, lv_ref, lo_ref, hi_ref, o_ref):
    i = pl.program_id(0)
    base = (tb_ref[i] // TILE_E) * TILE_E
    rel = dst_ref[...] - base
    sel = lv_ref[...] == 0
    hi_idx = jnp.where(sel, rel // R, -1)
    lo_idx = rel % R
    cols8 = lax.broadcasted_iota(jnp.int32, (TILE_E, 8), 1)
    cols128 = lax.broadcasted_iota(jnp.int32, (TILE_E, R), 1)
    a = (hi_idx == cols8).astype(jnp.float32)
    b = (lo_idx == cols128).astype(jnp.float32)
    w8 = jnp.concatenate([lo_ref[0], hi_ref[0]], axis=0)
    w8 = jnp.where(jnp.isfinite(w8), w8, 0.0)
    cvals = lax.dot_general(b, w8, (((1,), (1,)), ((), ())),
                            preferred_element_type=jnp.float32,
                            precision=lax.Precision.HIGHEST)
    o_ref[...] = jnp.sum(a * cvals, axis=1, keepdims=True)


def _expand(tb, dstf, lvf, ocr):
    grid_spec = pltpu.PrefetchScalarGridSpec(
        num_scalar_prefetch=1,
        grid=(N // TILE_E,),
        in_specs=[
            pl.BlockSpec((TILE_E, 1), lambda i, tb: (i, 0)),
            pl.BlockSpec((TILE_E, 1), lambda i, tb: (i, 0)),
            pl.BlockSpec((1, 4, R), lambda i, tb: (tb[i] // TILE_E, 0, 0)),
            pl.BlockSpec((1, 4, R), lambda i, tb: (tb[i] // TILE_E + 1, 0, 0)),
        ],
        out_specs=pl.BlockSpec((TILE_E, 1), lambda i, tb: (i, 0)),
    )
    return pl.pallas_call(
        _expand_body,
        grid_spec=grid_spec,
        out_shape=jax.ShapeDtypeStruct((N, 1), jnp.float32),
    )(tb, dstf, lvf, ocr, ocr)


def kernel(x, levels, W1, b1, W2, b2, W3, b3):
    lv2d = levels.astype(jnp.int32).reshape(R, R)
    w1b = W1.astype(jnp.bfloat16)
    w2b = W2.astype(jnp.bfloat16)

    dst, k1, opad = _route(lv2d)
    xc = _gather_rows(levels.astype(jnp.int32), x)
    oc = _mlp(k1, xc, w1b, b1, w2b, b2, W3, b3)
    ocr = oc.reshape((N + TILE_E) // TILE_E, 4, R)
    tb = opad[:, 0]
    res = _expand(tb, dst.reshape(N, 1), lv2d.reshape(N, 1), ocr)
    return res


# speedup vs baseline: 1.0850x; 1.0850x over previous
import dataclasses
import functools

import jax
import jax.numpy as jnp
from jax import lax
from jax.experimental import pallas as pl
from jax.experimental.pallas import tpu as pltpu
from jax.experimental.pallas import tpu_sc as plsc

N = 16384
D_IN = 2048
D_H = 1024
TILE_M = 512
TILE_E = 512
R = 128
NW = 32
ROWS_PW = R // NW
RPW = TILE_E
CH = 16
MAXC_PW = RPW // CH


def _mesh():
    return plsc.VectorSubcoreMesh(core_axis_name="c", subcore_axis_name="s")


def _sc_params():
    cp = pltpu.CompilerParams()
    if "needs_layout_passes" in pltpu.CompilerParams.__dataclass_fields__:
        cp = dataclasses.replace(cp, needs_layout_passes=False)
    return cp


def _dot(a, b):
    return lax.dot_general(a, b, (((1,), (0,)), ((), ())),
                           preferred_element_type=jnp.float32)


def _route_body(lv_ref, dst_ref, k1_ref, opad_ref):
    lv = lv_ref[...]
    m = (lv == 0).astype(jnp.float32)
    r = lax.broadcasted_iota(jnp.int32, (R, R), 0)
    c = lax.broadcasted_iota(jnp.int32, (R, R), 1)
    upper = (r < c).astype(jnp.float32)
    lower = (c < r).astype(jnp.float32)
    e = _dot(m, upper)
    s = e[:, R - 1:R] + m[:, R - 1:R]
    off = _dot(lower, s)

    wg = lax.broadcasted_iota(jnp.int32, (NW, R), 0)
    rg = lax.broadcasted_iota(jnp.int32, (NW, R), 1)
    grp = (rg // ROWS_PW == wg).astype(jnp.float32)
    cw = (_dot(grp, s) + 0.5).astype(jnp.int32)
    cpad16 = ((cw + 15) // 16).astype(jnp.float32)
    kpad = 16 * jnp.sum(cpad16).astype(jnp.int32)

    rw = lax.broadcasted_iota(jnp.int32, (R, NW), 0)
    ww = lax.broadcasted_iota(jnp.int32, (R, NW), 1)
    expl = (ww < rw // ROWS_PW).astype(jnp.float32)
    opad128 = 16.0 * _dot(expl, cpad16)
    lower4 = (c < (r // ROWS_PW) * ROWS_PW).astype(jnp.float32)
    off4 = _dot(lower4, s)
    dd = (opad128 - off4 + 0.5).astype(jnp.int32)
    pos = (e + off + 0.5).astype(jnp.int32) + dd
    dst_ref[...] = jnp.where(lv == 0, pos, N)
    k1_ref[0] = kpad

    w1 = lax.broadcasted_iota(jnp.int32, (NW, NW), 0)
    w2 = lax.broadcasted_iota(jnp.int32, (NW, NW), 1)
    lw32 = (w2 < w1).astype(jnp.float32)
    opad_ref[...] = (16.0 * _dot(lw32, cpad16) + 0.5).astype(jnp.int32)


def _route(lv2d):
    return pl.pallas_call(
        _route_body,
        out_shape=(
            jax.ShapeDtypeStruct((R, R), jnp.int32),
            jax.ShapeDtypeStruct((1,), jnp.int32),
            jax.ShapeDtypeStruct((NW, 1), jnp.int32),
        ),
        out_specs=(
            pl.BlockSpec((R, R), lambda: (0, 0)),
            pl.BlockSpec(memory_space=pltpu.SMEM),
            pl.BlockSpec((NW, 1), lambda: (0, 0)),
        ),
    )(lv2d)


def _gather_rows(dst, x):
  f = functools.partial(
      pl.kernel,
      out_type=jax.ShapeDtypeStruct((N, D_IN), jnp.float32),
      mesh=_mesh(),
      compiler_params=_sc_params(),
      scratch_types=[
          pltpu.VMEM((2 * ROWS_PW, R), jnp.int32),
          pltpu.VMEM((RPW,), jnp.int32),
          pltpu.VMEM((CH,), jnp.int32),
          pltpu.VMEM((CH,), jnp.int32),
          pltpu.VMEM((CH, D_IN), jnp.float32),
          pltpu.VMEM((CH, D_IN), jnp.float32),
          pltpu.SemaphoreType.DMA,
          pltpu.SemaphoreType.DMA,
          pltpu.SemaphoreType.DMA,
          pltpu.SemaphoreType.DMA,
      ],
  )

  @f
  def body(dst_hbm, x_hbm, xc_hbm, dst_v, lw, iv0, iv1, rows0,
           rows1, sg0, sg1, sw0, sw1):
    wid = lax.axis_index("s") * 2 + lax.axis_index("c")
    pltpu.sync_copy(dst_hbm.at[pl.ds((wid // 2) * 2 * ROWS_PW, 2 * ROWS_PW)],
                    dst_v)
    half = (wid % 2) * ROWS_PW
    lane = lax.iota(jnp.int32, 16)

    def grp(g, carry):
        cnt, mn = carry
        d16 = dst_v[half + g // 8, pl.ds((g % 8) * 16, 16)]
        msk = d16 < N
        ids = wid * RPW + g * 16 + lane
        plsc.store_compressed(lw.at[pl.ds(cnt, 16)], ids, mask=msk)
        return (cnt + jnp.sum(msk.astype(jnp.int32), axis=0),
                jnp.minimum(mn, d16))

    cnt, mn = lax.fori_loop(
        0, 32, grp, (jnp.int32(0), jnp.full((16,), N, jnp.int32)))
    p16 = jnp.min(mn, axis=0) // CH
    ivs = (iv0, iv1)
    rows = (rows0, rows1)
    sgs = (sg0, sg1)
    sws = (sw0, sw1)

    @pl.loop(0, MAXC_PW, step=2)
    def _pair(cl):
        for b in range(2):
            @pl.when((cl + b) * CH < cnt)
            def _(b=b):
                c = cl + b
                ivs[b][...] = plsc.load_gather(
                    lw, [jnp.minimum(c * CH + lane, cnt - 1)])
                pltpu.make_async_copy(x_hbm.at[ivs[b]], rows[b],
                                      sgs[b]).start()
        for b in range(2):
            @pl.when((cl + b) * CH < cnt)
            def _(b=b):
                c = cl + b
                pltpu.make_async_copy(x_hbm.at[ivs[b]], rows[b],
                                      sgs[b]).wait()
                pltpu.make_async_copy(
                    rows[b], xc_hbm.at[pl.ds((p16 + c) * CH, CH)],
                    sws[b]).start()
        for b in range(2):
            @pl.when((cl + b) * CH < cnt)
            def _(b=b):
                c = cl + b
                pltpu.make_async_copy(
                    rows[b], xc_hbm.at[pl.ds((p16 + c) * CH, CH)],
                    sws[b]).wait()

  return body(dst, x)


def _mlp_body(k_ref, x_ref, w1_ref, b1_ref, w2_ref, b2_ref, w3_ref, b3_ref,
              o_ref):
    nt = (k_ref[0] + TILE_M - 1) // TILE_M

    @pl.when(pl.program_id(0) < nt)
    def _():
        xb = x_ref[...].astype(jnp.bfloat16)
        h1 = lax.dot_general(xb, w1_ref[...], (((1,), (0,)), ((), ())),
                             preferred_element_type=jnp.float32)
        h1 = jnp.maximum(h1 + b1_ref[...][None, :], 0.0).astype(jnp.bfloat16)
        h2 = lax.dot_general(h1, w2_ref[...], (((1,), (0,)), ((), ())),
                             preferred_element_type=jnp.float32)
        h2 = jnp.maximum(h2 + b2_ref[...][None, :], 0.0)
        out = lax.dot_general(h2, w3_ref[...], (((1,), (0,)), ((), ())),
                              preferred_element_type=jnp.float32)
        o_ref[...] = out + b3_ref[...][None, :]


def _mlp(k1, xc, w1b, b1, w2b, b2, W3, b3):
    def _x_map(i, kr):
        nt = (kr[0] + TILE_M - 1) // TILE_M
        return (jnp.minimum(i, jnp.maximum(nt - 1, 0)), 0)

    grid_spec = pltpu.PrefetchScalarGridSpec(
        num_scalar_prefetch=1,
        grid=(N // TILE_M,),
        in_specs=[
            pl.BlockSpec((TILE_M, D_IN), _x_map),
            pl.BlockSpec((D_IN, D_H), lambda i, kr: (0, 0)),
            pl.BlockSpec((D_H,), lambda i, kr: (0,)),
            pl.BlockSpec((D_H, D_H), lambda i, kr: (0, 0)),
            pl.BlockSpec((D_H,), lambda i, kr: (0,)),
            pl.BlockSpec((D_H, 1), lambda i, kr: (0, 0)),
            pl.BlockSpec((1,), lambda i, kr: (0,)),
        ],
        out_specs=pl.BlockSpec((TILE_M, 1), lambda i, kr: (i, 0)),
    )
    return pl.pallas_call(
        _mlp_body,
        grid_spec=grid_spec,
        out_shape=jax.ShapeDtypeStruct((N + TILE_E, 1), jnp.float32),
    )(k1, xc, w1b, b1, w2b, b2, W3, b3)


def _expand_body(tb_ref, dst_ref, lv_ref, lo_ref, hi_ref, o_ref):
    i = pl.program_id(0)
    base = (tb_ref[i] // TILE_E) * TILE_E
    rel = dst_ref[...] - base
    sel = lv_ref[...] == 0
    hi_idx = jnp.where(sel, rel // R, -1)
    lo_idx = rel % R
    cols8 = lax.broadcasted_iota(jnp.int32, (TILE_E, 8), 1)
    cols128 = lax.broadcasted_iota(jnp.int32, (TILE_E, R), 1)
    a = (hi_idx == cols8).astype(jnp.float32)
    b = (lo_idx == cols128).astype(jnp.float32)
    w8 = jnp.concatenate([lo_ref[0], hi_ref[0]], axis=0)
    w8 = jnp.where(jnp.isfinite(w8), w8, 0.0)
    cvals = lax.dot_general(b, w8, (((1,), (1,)), ((), ())),
                            preferred_element_type=jnp.float32,
                            precision=lax.Precision.HIGHEST)
    o_ref[...] = jnp.sum(a * cvals, axis=1, keepdims=True)


def _expand(tb, dstf, lvf, ocr):
    grid_spec = pltpu.PrefetchScalarGridSpec(
        num_scalar_prefetch=1,
        grid=(N // TILE_E,),
        in_specs=[
            pl.BlockSpec((TILE_E, 1), lambda i, tb: (i, 0)),
            pl.BlockSpec((TILE_E, 1), lambda i, tb: (i, 0)),
            pl.BlockSpec((1, 4, R), lambda i, tb: (tb[i] // TILE_E, 0, 0)),
            pl.BlockSpec((1, 4, R), lambda i, tb: (tb[i] // TILE_E + 1, 0, 0)),
        ],
        out_specs=pl.BlockSpec((TILE_E, 1), lambda i, tb: (i, 0)),
    )
    return pl.pallas_call(
        _expand_body,
        grid_spec=grid_spec,
        out_shape=jax.ShapeDtypeStruct((N, 1), jnp.float32),
    )(tb, dstf, lvf, ocr, ocr)


def kernel(x, levels, W1, b1, W2, b2, W3, b3):
    lv2d = levels.astype(jnp.int32).reshape(R, R)
    w1b = W1.astype(jnp.bfloat16)
    w2b = W2.astype(jnp.bfloat16)

    dst, k1, opad = _route(lv2d)
    xc = _gather_rows(dst, x)
    oc = _mlp(k1, xc, w1b, b1, w2b, b2, W3, b3)
    ocr = oc.reshape((N + TILE_E) // TILE_E, 4, R)
    tb = opad[:, 0]
    res = _expand(tb, dst.reshape(N, 1), lv2d.reshape(N, 1), ocr)
    return res


# speedup vs baseline: 1.0932x; 1.0076x over previous
import dataclasses
import functools

import jax
import jax.numpy as jnp
from jax import lax
from jax.experimental import pallas as pl
from jax.experimental.pallas import tpu as pltpu
from jax.experimental.pallas import tpu_sc as plsc

N = 16384
D_IN = 2048
D_H = 1024
TILE_M = 1024
TILE_E = 512
R = 128
NW = 32
ROWS_PW = R // NW
RPW = TILE_E
CH = 16
MAXC_PW = RPW // CH


def _mesh():
    return plsc.VectorSubcoreMesh(core_axis_name="c", subcore_axis_name="s")


def _sc_params():
    cp = pltpu.CompilerParams()
    if "needs_layout_passes" in pltpu.CompilerParams.__dataclass_fields__:
        cp = dataclasses.replace(cp, needs_layout_passes=False)
    return cp


def _dot(a, b):
    return lax.dot_general(a, b, (((1,), (0,)), ((), ())),
                           preferred_element_type=jnp.float32)


def _route_body(lv_ref, dst_ref, k1_ref, opad_ref):
    lv = lv_ref[...]
    m = (lv == 0).astype(jnp.float32)
    r = lax.broadcasted_iota(jnp.int32, (R, R), 0)
    c = lax.broadcasted_iota(jnp.int32, (R, R), 1)
    upper = (r < c).astype(jnp.float32)
    lower = (c < r).astype(jnp.float32)
    e = _dot(m, upper)
    s = e[:, R - 1:R] + m[:, R - 1:R]
    off = _dot(lower, s)

    wg = lax.broadcasted_iota(jnp.int32, (NW, R), 0)
    rg = lax.broadcasted_iota(jnp.int32, (NW, R), 1)
    grp = (rg // ROWS_PW == wg).astype(jnp.float32)
    cw = (_dot(grp, s) + 0.5).astype(jnp.int32)
    cpad16 = ((cw + 15) // 16).astype(jnp.float32)
    kpad = 16 * jnp.sum(cpad16).astype(jnp.int32)

    rw = lax.broadcasted_iota(jnp.int32, (R, NW), 0)
    ww = lax.broadcasted_iota(jnp.int32, (R, NW), 1)
    expl = (ww < rw // ROWS_PW).astype(jnp.float32)
    opad128 = 16.0 * _dot(expl, cpad16)
    lower4 = (c < (r // ROWS_PW) * ROWS_PW).astype(jnp.float32)
    off4 = _dot(lower4, s)
    dd = (opad128 - off4 + 0.5).astype(jnp.int32)
    pos = (e + off + 0.5).astype(jnp.int32) + dd
    dst_ref[...] = jnp.where(lv == 0, pos, N)
    k1_ref[0] = kpad

    w1 = lax.broadcasted_iota(jnp.int32, (NW, NW), 0)
    w2 = lax.broadcasted_iota(jnp.int32, (NW, NW), 1)
    lw32 = (w2 < w1).astype(jnp.float32)
    opad_ref[...] = (16.0 * _dot(lw32, cpad16) + 0.5).astype(jnp.int32)


def _route(lv2d):
    return pl.pallas_call(
        _route_body,
        out_shape=(
            jax.ShapeDtypeStruct((R, R), jnp.int32),
            jax.ShapeDtypeStruct((1,), jnp.int32),
            jax.ShapeDtypeStruct((NW, 1), jnp.int32),
        ),
        out_specs=(
            pl.BlockSpec((R, R), lambda: (0, 0)),
            pl.BlockSpec(memory_space=pltpu.SMEM),
            pl.BlockSpec((NW, 1), lambda: (0, 0)),
        ),
    )(lv2d)


def _gather_rows(dst, x):
  f = functools.partial(
      pl.kernel,
      out_type=jax.ShapeDtypeStruct((N, D_IN), jnp.float32),
      mesh=_mesh(),
      compiler_params=_sc_params(),
      scratch_types=[
          pltpu.VMEM((2 * ROWS_PW, R), jnp.int32),
          pltpu.VMEM((RPW,), jnp.int32),
          pltpu.VMEM((CH,), jnp.int32),
          pltpu.VMEM((CH,), jnp.int32),
          pltpu.VMEM((CH, D_IN), jnp.float32),
          pltpu.VMEM((CH, D_IN), jnp.float32),
          pltpu.SemaphoreType.DMA,
          pltpu.SemaphoreType.DMA,
          pltpu.SemaphoreType.DMA,
          pltpu.SemaphoreType.DMA,
      ],
  )

  @f
  def body(dst_hbm, x_hbm, xc_hbm, dst_v, lw, iv0, iv1, rows0,
           rows1, sg0, sg1, sw0, sw1):
    wid = lax.axis_index("s") * 2 + lax.axis_index("c")
    pltpu.sync_copy(dst_hbm.at[pl.ds((wid // 2) * 2 * ROWS_PW, 2 * ROWS_PW)],
                    dst_v)
    half = (wid % 2) * ROWS_PW
    lane = lax.iota(jnp.int32, 16)

    def grp(g, carry):
        cnt, mn = carry
        d16 = dst_v[half + g // 8, pl.ds((g % 8) * 16, 16)]
        msk = d16 < N
        ids = wid * RPW + g * 16 + lane
        plsc.store_compressed(lw.at[pl.ds(cnt, 16)], ids, mask=msk)
        return (cnt + jnp.sum(msk.astype(jnp.int32), axis=0),
                jnp.minimum(mn, d16))

    cnt, mn = lax.fori_loop(
        0, 32, grp, (jnp.int32(0), jnp.full((16,), N, jnp.int32)))
    p16 = jnp.min(mn, axis=0) // CH
    ivs = (iv0, iv1)
    rows = (rows0, rows1)
    sgs = (sg0, sg1)
    sws = (sw0, sw1)

    @pl.loop(0, MAXC_PW, step=2)
    def _pair(cl):
        for b in range(2):
            @pl.when((cl + b) * CH < cnt)
            def _(b=b):
                c = cl + b
                ivs[b][...] = plsc.load_gather(
                    lw, [jnp.minimum(c * CH + lane, cnt - 1)])
                pltpu.make_async_copy(x_hbm.at[ivs[b]], rows[b],
                                      sgs[b]).start()
        for b in range(2):
            @pl.when((cl + b) * CH < cnt)
            def _(b=b):
                c = cl + b
                pltpu.make_async_copy(x_hbm.at[ivs[b]], rows[b],
                                      sgs[b]).wait()
                pltpu.make_async_copy(
                    rows[b], xc_hbm.at[pl.ds((p16 + c) * CH, CH)],
                    sws[b]).start()
        for b in range(2):
            @pl.when((cl + b) * CH < cnt)
            def _(b=b):
                c = cl + b
                pltpu.make_async_copy(
                    rows[b], xc_hbm.at[pl.ds((p16 + c) * CH, CH)],
                    sws[b]).wait()

  return body(dst, x)


def _mlp_body(k_ref, x_ref, w1_ref, b1_ref, w2_ref, b2_ref, w3_ref, b3_ref,
              o_ref):
    nt = (k_ref[0] + TILE_M - 1) // TILE_M

    @pl.when(pl.program_id(0) < nt)
    def _():
        xb = x_ref[...].astype(jnp.bfloat16)
        h1 = lax.dot_general(xb, w1_ref[...], (((1,), (0,)), ((), ())),
                             preferred_element_type=jnp.float32)
        h1 = jnp.maximum(h1 + b1_ref[...][None, :], 0.0).astype(jnp.bfloat16)
        h2 = lax.dot_general(h1, w2_ref[...], (((1,), (0,)), ((), ())),
                             preferred_element_type=jnp.float32)
        h2 = jnp.maximum(h2 + b2_ref[...][None, :], 0.0)
        out = lax.dot_general(h2, w3_ref[...], (((1,), (0,)), ((), ())),
                              preferred_element_type=jnp.float32)
        o_ref[...] = out + b3_ref[...][None, :]


def _mlp(k1, xc, w1b, b1, w2b, b2, W3, b3):
    def _x_map(i, kr):
        nt = (kr[0] + TILE_M - 1) // TILE_M
        return (jnp.minimum(i, jnp.maximum(nt - 1, 0)), 0)

    grid_spec = pltpu.PrefetchScalarGridSpec(
        num_scalar_prefetch=1,
        grid=(N // TILE_M,),
        in_specs=[
            pl.BlockSpec((TILE_M, D_IN), _x_map),
            pl.BlockSpec((D_IN, D_H), lambda i, kr: (0, 0)),
            pl.BlockSpec((D_H,), lambda i, kr: (0,)),
            pl.BlockSpec((D_H, D_H), lambda i, kr: (0, 0)),
            pl.BlockSpec((D_H,), lambda i, kr: (0,)),
            pl.BlockSpec((D_H, 1), lambda i, kr: (0, 0)),
            pl.BlockSpec((1,), lambda i, kr: (0,)),
        ],
        out_specs=pl.BlockSpec((TILE_M, 1), lambda i, kr: (i, 0)),
    )
    return pl.pallas_call(
        _mlp_body,
        grid_spec=grid_spec,
        out_shape=jax.ShapeDtypeStruct((N + TILE_E, 1), jnp.float32),
    )(k1, xc, w1b, b1, w2b, b2, W3, b3)


def _expand_body(tb_ref, dst_ref, lv_ref, lo_ref, hi_ref, o_ref):
    i = pl.program_id(0)
    base = (tb_ref[i] // TILE_E) * TILE_E
    rel = dst_ref[...] - base
    sel = lv_ref[...] == 0
    hi_idx = jnp.where(sel, rel // R, -1)
    lo_idx = rel % R
    cols8 = lax.broadcasted_iota(jnp.int32, (TILE_E, 8), 1)
    cols128 = lax.broadcasted_iota(jnp.int32, (TILE_E, R), 1)
    a = (hi_idx == cols8).astype(jnp.float32)
    b = (lo_idx == cols128).astype(jnp.float32)
    w8 = jnp.concatenate([lo_ref[0], hi_ref[0]], axis=0)
    w8 = jnp.where(jnp.isfinite(w8), w8, 0.0)
    cvals = lax.dot_general(b, w8, (((1,), (1,)), ((), ())),
                            preferred_element_type=jnp.float32,
                            precision=lax.Precision.HIGHEST)
    o_ref[...] = jnp.sum(a * cvals, axis=1, keepdims=True)


def _expand(tb, dstf, lvf, ocr):
    grid_spec = pltpu.PrefetchScalarGridSpec(
        num_scalar_prefetch=1,
        grid=(N // TILE_E,),
        in_specs=[
            pl.BlockSpec((TILE_E, 1), lambda i, tb: (i, 0)),
            pl.BlockSpec((TILE_E, 1), lambda i, tb: (i, 0)),
            pl.BlockSpec((1, 4, R), lambda i, tb: (tb[i] // TILE_E, 0, 0)),
            pl.BlockSpec((1, 4, R), lambda i, tb: (tb[i] // TILE_E + 1, 0, 0)),
        ],
        out_specs=pl.BlockSpec((TILE_E, 1), lambda i, tb: (i, 0)),
    )
    return pl.pallas_call(
        _expand_body,
        grid_spec=grid_spec,
        out_shape=jax.ShapeDtypeStruct((N, 1), jnp.float32),
    )(tb, dstf, lvf, ocr, ocr)


def kernel(x, levels, W1, b1, W2, b2, W3, b3):
    lv2d = levels.astype(jnp.int32).reshape(R, R)
    w1b = W1.astype(jnp.bfloat16)
    w2b = W2.astype(jnp.bfloat16)

    dst, k1, opad = _route(lv2d)
    xc = _gather_rows(dst, x)
    oc = _mlp(k1, xc, w1b, b1, w2b, b2, W3, b3)
    ocr = oc.reshape((N + TILE_E) // TILE_E, 4, R)
    tb = opad[:, 0]
    res = _expand(tb, dst.reshape(N, 1), lv2d.reshape(N, 1), ocr)
    return res
